# Initial kernel scaffold; baseline (speedup 1.0000x reference)
import jax
import jax.numpy as jnp
from jax.experimental import pallas as pl


def kernel(item_emb, bi_rows, bi_cols):
    raise NotImplementedError("write your pallas kernel here")


# speedup vs baseline: 6.2496x; 6.2496x over previous
import functools

import jax
import jax.numpy as jnp
from jax import lax
from jax.experimental import pallas as pl
from jax.experimental.pallas import tpu as pltpu
from jax.experimental.pallas import tpu_sc as plsc

N_ITEMS = 10000
N_HE = 5000
NNZ = 320000
D = 128

NC = 2
NS = 16
NW = NC * NS
CH = 128
NCHUNKS = NNZ // CH
CH_PER_W = NCHUNKS // NW
CH_REM = NCHUNKS - CH_PER_W * NW

H_PAD = 5120
N_PAD = 10240
WB = 160


def _spmm_sc(table_rows, s_pad, with_deg):
    rows_per_tile = s_pad // NS
    n_wb = rows_per_tile // WB
    dh_per_tile = H_PAD // NS
    di_per_tile = N_PAD // NS

    out_type = [jax.ShapeDtypeStruct((NC, s_pad, D), jnp.float32)]
    scratch = [
        pltpu.VMEM((CH,), jnp.int32),
        pltpu.VMEM((CH,), jnp.int32),
        pltpu.VMEM((CH, D), jnp.float32),
        pltpu.VMEM((WB, D), jnp.float32),
        pltpu.VMEM_SHARED((s_pad, D), jnp.float32),
        pltpu.SemaphoreType.DMA,
    ]
    if with_deg:
        out_type += [jax.ShapeDtypeStruct((NC * H_PAD,), jnp.float32),
                     jax.ShapeDtypeStruct((NC * N_PAD,), jnp.float32)]
        scratch += [
            pltpu.VMEM((CH,), jnp.float32),
            pltpu.VMEM((di_per_tile,), jnp.float32),
            pltpu.VMEM_SHARED((H_PAD,), jnp.float32),
            pltpu.VMEM_SHARED((N_PAD,), jnp.float32),
        ]

    mesh = plsc.VectorSubcoreMesh(core_axis_name="c", subcore_axis_name="s")

    def body(table, gidx, sidx, zeros2d, zeros1d, ones, *rest):
        if with_deg:
            (p_out, dh_out, di_out,
             gbuf, sbuf, rbuf, zbuf, acc, sem, obuf, dzbuf, dh_acc, di_acc) = rest
        else:
            (p_out, gbuf, sbuf, rbuf, zbuf, acc, sem) = rest
        c = lax.axis_index("c")
        s = lax.axis_index("s")
        wid = s * NC + c

        pltpu.sync_copy(zeros2d, zbuf)
        row0 = s * rows_per_tile
        for k in range(n_wb):
            pltpu.sync_copy(zbuf, acc.at[pl.ds(row0 + k * WB, WB), :])
        if with_deg:
            pltpu.sync_copy(ones, obuf)
            pltpu.sync_copy(zeros1d, dzbuf)
            pltpu.sync_copy(dzbuf.at[pl.ds(0, dh_per_tile)],
                            dh_acc.at[pl.ds(s * dh_per_tile, dh_per_tile)])
            pltpu.sync_copy(dzbuf.at[pl.ds(0, di_per_tile)],
                            di_acc.at[pl.ds(s * di_per_tile, di_per_tile)])
        plsc.subcore_barrier()

        def do_chunk(ch):
            pltpu.sync_copy(gidx.at[ch], gbuf)
            pltpu.sync_copy(sidx.at[ch], sbuf)
            pltpu.async_copy(table.at[gbuf], rbuf, sem).wait()
            pltpu.sync_copy(rbuf, acc.at[sbuf], add=True)
            if with_deg:
                pltpu.sync_copy(obuf, dh_acc.at[sbuf], add=True)
                pltpu.sync_copy(obuf, di_acc.at[gbuf], add=True)

        def loop_body(j, carry):
            do_chunk(wid * CH_PER_W + j)
            return carry

        lax.fori_loop(0, CH_PER_W, loop_body, 0)

        @pl.when(wid < CH_REM)
        def _():
            do_chunk(NW * CH_PER_W + wid)

        plsc.subcore_barrier()

        for k in range(n_wb):
            r0 = row0 + k * WB
            pltpu.sync_copy(acc.at[pl.ds(r0, WB), :], zbuf)
            pltpu.sync_copy(zbuf, p_out.at[c, pl.ds(r0, WB), :])
        if with_deg:
            hr0 = s * dh_per_tile
            pltpu.sync_copy(dh_acc.at[pl.ds(hr0, dh_per_tile)],
                            dzbuf.at[pl.ds(0, dh_per_tile)])
            pltpu.sync_copy(dzbuf.at[pl.ds(0, dh_per_tile)],
                            dh_out.at[pl.ds(c * H_PAD + hr0, dh_per_tile)])
            ir0 = s * di_per_tile
            pltpu.sync_copy(di_acc.at[pl.ds(ir0, di_per_tile)], dzbuf)
            pltpu.sync_copy(dzbuf, di_out.at[pl.ds(c * N_PAD + ir0, di_per_tile)])

    return pl.kernel(body, out_type=out_type, mesh=mesh,
                     scratch_types=scratch)


def _combine_tc(s_pad):
    blk = 512
    grid = (s_pad // blk,)

    def body(p_ref, d_ref, o_ref):
        deg = d_ref[0] + d_ref[1]
        dinv = jnp.where(deg > 0.0, 1.0 / jnp.where(deg > 0.0, deg, 1.0), 0.0)
        o_ref[...] = (p_ref[0] + p_ref[1]) * dinv

    return pl.pallas_call(
        body,
        grid=grid,
        in_specs=[
            pl.BlockSpec((2, blk, D), lambda i: (0, i, 0)),
            pl.BlockSpec((2, blk, 1), lambda i: (0, i, 0)),
        ],
        out_specs=pl.BlockSpec((blk, D), lambda i: (i, 0)),
        out_shape=jax.ShapeDtypeStruct((s_pad, D), jnp.float32),
    )


def _final_tc():
    blk = 400
    grid = (N_ITEMS // blk,)

    def body(x0_ref, x1_ref, p_ref, d_ref, o_ref):
        deg = d_ref[0] + d_ref[1]
        dinv = jnp.where(deg > 0.0, 1.0 / jnp.where(deg > 0.0, deg, 1.0), 0.0)
        x2 = (p_ref[0] + p_ref[1]) * dinv
        m = (x0_ref[...] + x1_ref[...] + x2) * (1.0 / 3.0)
        nrm = jnp.sqrt(jnp.sum(m * m, axis=1, keepdims=True))
        o_ref[...] = m / jnp.maximum(nrm, 1e-12)

    return pl.pallas_call(
        body,
        grid=grid,
        in_specs=[
            pl.BlockSpec((blk, D), lambda i: (i, 0)),
            pl.BlockSpec((blk, D), lambda i: (i, 0)),
            pl.BlockSpec((2, blk, D), lambda i: (0, i, 0)),
            pl.BlockSpec((2, blk, 1), lambda i: (0, i, 0)),
        ],
        out_specs=pl.BlockSpec((blk, D), lambda i: (i, 0)),
        out_shape=jax.ShapeDtypeStruct((N_ITEMS, D), jnp.float32),
    )


@jax.jit
def kernel(item_emb, bi_rows, bi_cols):
    gidx = bi_cols.reshape(NCHUNKS, CH)
    sidx = bi_rows.reshape(NCHUNKS, CH)
    zeros2d = jnp.zeros((WB, D), jnp.float32)
    zeros1d = jnp.zeros((N_PAD // NS,), jnp.float32)
    ones = jnp.ones((CH,), jnp.float32)

    mp1, dhp, dip = _spmm_sc(N_ITEMS, H_PAD, True)(
        item_emb, gidx, sidx, zeros2d, zeros1d, ones)
    dhp3 = dhp.reshape(NC, H_PAD, 1)
    dip3 = dip.reshape(NC, N_PAD, 1)
    msg1 = _combine_tc(H_PAD)(mp1, dhp3)

    (xp1,) = _spmm_sc(H_PAD, N_PAD, False)(
        msg1, sidx, gidx, zeros2d, zeros1d, ones)
    x1 = _combine_tc(N_PAD)(xp1, dip3)

    (mp2,) = _spmm_sc(N_PAD, H_PAD, False)(
        x1, gidx, sidx, zeros2d, zeros1d, ones)
    msg2 = _combine_tc(H_PAD)(mp2, dhp3)

    (xp2,) = _spmm_sc(H_PAD, N_PAD, False)(
        msg2, sidx, gidx, zeros2d, zeros1d, ones)

    return _final_tc()(item_emb, x1, xp2, dip3)
